# baseline (speedup 1.0000x reference)
import dataclasses
import functools

import jax
import jax.numpy as jnp
from jax import lax
from jax.experimental import pallas as pl
from jax.experimental.pallas import tpu as pltpu
from jax.experimental.pallas import tpu_sc as plsc

import numpy as np

_NH, _NL, _NP, _HD = 8, 4, 4, 32
_UNPACK_PERM = np.array(
    [h * 32 + (2 * m if m < 16 else 2 * (m - 16) + 1)
     for h in range(8) for m in range(32)], dtype=np.int32)
_SHAPES = ((64, 64), (32, 32), (16, 16), (8, 8))
_NT = sum(h * w for h, w in _SHAPES)
_BASES = (0, 4096, 5120, 5376)
_NW = 32
_TAPS = _NL * _NP * 4
_HP = lax.Precision.HIGHEST


def _matmul_bias_kernel(x_ref, w_ref, b_ref, o_ref):
    o_ref[...] = (
        jnp.dot(x_ref[...], w_ref[...], preferred_element_type=jnp.float32,
                precision=_HP)
        + b_ref[...]
    ).astype(o_ref.dtype)


def _project(x, w, b, block_rows, out_dtype=jnp.float32):
    n, d = x.shape
    dout = w.shape[1]
    return pl.pallas_call(
        _matmul_bias_kernel,
        grid=(n // block_rows,),
        in_specs=[
            pl.BlockSpec((block_rows, d), lambda i: (i, 0)),
            pl.BlockSpec((d, dout), lambda i: (0, 0)),
            pl.BlockSpec((1, dout), lambda i: (0, 0)),
        ],
        out_specs=pl.BlockSpec((block_rows, dout), lambda i: (i, 0)),
        out_shape=jax.ShapeDtypeStruct((n, dout), out_dtype),
    )(x, w, b.reshape(1, dout))


def _sampling_math(q, rx, ry, wso, bso, waw, baw, lq):
    f32 = jnp.float32
    nq = q.shape[0]
    so = jnp.dot(q, wso, preferred_element_type=f32, precision=_HP) + bso
    rr = lax.broadcasted_iota(jnp.int32, (256, 128), 0)
    cc = lax.broadcasted_iota(jnp.int32, (256, 128), 1)
    selx = (rr == 2 * cc).astype(f32)
    sely = (rr == 2 * cc + 1).astype(f32)
    sox = jnp.dot(so, selx, preferred_element_type=f32, precision=_HP)
    soy = jnp.dot(so, sely, preferred_element_type=f32, precision=_HP)
    logits = jnp.dot(q, waw, preferred_element_type=f32, precision=_HP) + baw
    z = jnp.exp(logits)
    r128 = lax.broadcasted_iota(jnp.int32, (128, 128), 0)
    c128 = lax.broadcasted_iota(jnp.int32, (128, 128), 1)
    bd = (r128 // (_NL * _NP) == c128 // (_NL * _NP)).astype(f32)
    aw = z / jnp.dot(z, bd, preferred_element_type=f32, precision=_HP)

    shape = (nq, 128)
    lane = lax.broadcasted_iota(jnp.int32, shape, 1)
    row = lax.broadcasted_iota(jnp.int32, shape, 0)
    lvl = (lane // _NP) % _NL
    h_i = lane // (_NL * _NP)
    b_i = row // lq
    wf = jnp.where(lvl == 0, 64.0, jnp.where(lvl == 1, 32.0,
                   jnp.where(lvl == 2, 16.0, 8.0)))
    base = jnp.where(lvl == 0, _BASES[0], jnp.where(lvl == 1, _BASES[1],
                     jnp.where(lvl == 2, _BASES[2], _BASES[3])))
    cx = jnp.where(lvl == 0, rx[:, 0:1], jnp.where(lvl == 1, rx[:, 1:2],
                   jnp.where(lvl == 2, rx[:, 2:3], rx[:, 3:4])))
    cy = jnp.where(lvl == 0, ry[:, 0:1], jnp.where(lvl == 1, ry[:, 1:2],
                   jnp.where(lvl == 2, ry[:, 2:3], ry[:, 3:4])))
    locx = cx + sox / wf
    locy = cy + soy / wf
    gx = 2.0 * locx - 1.0
    gy = 2.0 * locy - 1.0
    ix = ((gx + 1.0) * wf - 1.0) * 0.5
    iy = ((gy + 1.0) * wf - 1.0) * 0.5
    ix0f = jnp.floor(ix)
    iy0f = jnp.floor(iy)
    fx = ix - ix0f
    fy = iy - iy0f
    ix0 = ix0f.astype(jnp.int32)
    iy0 = iy0f.astype(jnp.int32)
    ix1 = ix0 + 1
    iy1 = iy0 + 1
    wi = wf.astype(jnp.int32)
    vx0 = (ix0 >= 0) & (ix0 < wi)
    vx1 = (ix1 >= 0) & (ix1 < wi)
    vy0 = (iy0 >= 0) & (iy0 < wi)
    vy1 = (iy1 >= 0) & (iy1 < wi)
    m00 = (vx0 & vy0).astype(f32)
    m01 = (vx1 & vy0).astype(f32)
    m10 = (vx0 & vy1).astype(f32)
    m11 = (vx1 & vy1).astype(f32)
    ix0c = jnp.clip(ix0, 0, wi - 1)
    ix1c = jnp.clip(ix1, 0, wi - 1)
    iy0c = jnp.clip(iy0, 0, wi - 1)
    iy1c = jnp.clip(iy1, 0, wi - 1)
    rowbase = b_i * (_NT * _NH) + base * _NH + h_i
    i00 = rowbase + (iy0c * wi + ix0c) * _NH
    i01 = rowbase + (iy0c * wi + ix1c) * _NH
    i10 = rowbase + (iy1c * wi + ix0c) * _NH
    i11 = rowbase + (iy1c * wi + ix1c) * _NH
    ex = 1.0 - fx
    ey = 1.0 - fy
    w00 = ex * ey * m00 * aw
    w01 = fx * ey * m01 * aw
    w10 = ex * fy * m10 * aw
    w11 = fx * fy * m11 * aw
    return i00, i01, i10, i11, w00, w01, w10, w11


def _sampling_kernel(q_ref, rx_ref, ry_ref, wso_ref, bso_ref, waw_ref,
                     baw_ref, *out_refs, lq):
    outs = _sampling_math(q_ref[...], rx_ref[...], ry_ref[...], wso_ref[...],
                          bso_ref[...], waw_ref[...], baw_ref[...], lq)
    for ref, val in zip(out_refs, outs):
        ref[...] = val


def _compute_taps(query, reference_points, W_so, b_so, W_aw, b_aw):
    b, lq, d = query.shape
    nq = b * lq
    q2 = query.reshape(nq, d)
    rx = reference_points[..., 0].reshape(nq, _NL)
    ry = reference_points[..., 1].reshape(nq, _NL)
    ishape = jax.ShapeDtypeStruct((nq, 128), jnp.int32)
    fshape = jax.ShapeDtypeStruct((nq, 128), jnp.float32)
    return pl.pallas_call(
        functools.partial(_sampling_kernel, lq=lq),
        out_shape=[ishape] * 4 + [fshape] * 4,
    )(q2, rx, ry, W_so, b_so.reshape(1, 256), W_aw, b_aw.reshape(1, 128))


def _sc_gather(table, idxs, wtss, opw):
    mesh = plsc.VectorSubcoreMesh(core_axis_name="c", subcore_axis_name="s")
    cp = pltpu.CompilerParams(use_tc_tiling_on_sc=False)
    if "needs_layout_passes" in pltpu.CompilerParams.__dataclass_fields__:
        cp = dataclasses.replace(cp, needs_layout_passes=False)

    nbuf = 3
    opc = 5
    epw = opw * 16
    nchunk = opw // opc
    bcast_dnums = lax.GatherDimensionNumbers(
        offset_dims=(), collapsed_slice_dims=(0,), start_index_map=(0,))

    def body(table_hbm, i0_h, i1_h, i2_h, i3_h, w0_h, w1_h, w2_h, w3_h,
             out_hbm, i0, i1, i2, i3, w0, w1, w2, w3, rbuf, obuf, *sems):
        wid = lax.axis_index("s") * 2 + lax.axis_index("c")
        base = wid * epw
        ibufs = (i0, i1, i2, i3)
        wbufs = (w0, w1, w2, w3)
        for src, dst in zip((i0_h, i1_h, i2_h, i3_h), ibufs):
            pltpu.sync_copy(src.at[pl.ds(base, epw)], dst)
        for src, dst in zip((w0_h, w1_h, w2_h, w3_h), wbufs):
            pltpu.sync_copy(src.at[pl.ds(base, epw)], dst)

        nrow = opc * 16

        def mkcopy(c, t, slot):
            return pltpu.make_async_copy(
                table_hbm.at[ibufs[t].at[pl.ds(c * nrow, nrow)]],
                rbuf.at[slot, pl.ds(t * nrow, nrow)], sems[slot])

        def start(c, slot):
            for t in range(4):
                mkcopy(c, t, slot).start()

        def wait(c, slot):
            for t in range(4):
                mkcopy(c, t, slot).wait()

        def compute(slot, c):
            for k in range(opc):
                o = c * opc + k
                acc0 = jnp.zeros((16,), jnp.float32)
                acc1 = jnp.zeros((16,), jnp.float32)
                for t in range(4):
                    wv = wbufs[t][pl.ds(o * 16, 16)]
                    for j in range(16):
                        wj = lax.gather(
                            wv, jnp.full((16, 1), j, jnp.int32), bcast_dnums,
                            (1,),
                            mode=lax.GatherScatterMode.PROMISE_IN_BOUNDS)
                        r = t * nrow + k * 16 + j
                        row = rbuf[slot, r, pl.ds(0, 32)]
                        v0, v1 = plsc.unpack(
                            row, format=plsc.PackFormat.INTERLEAVED)
                        acc0 = acc0 + wj * v0
                        acc1 = acc1 + wj * v1
                obuf[o, pl.ds(0, 16)] = acc0
                obuf[o, pl.ds(16, 16)] = acc1

        for s in range(nbuf):
            start(s, s)

        @pl.loop(0, nchunk // nbuf)
        def _(i):
            for s in range(nbuf):
                c = i * nbuf + s
                wait(c, s)
                compute(s, c)

                @pl.when(c + nbuf < nchunk)
                def _():
                    start(c + nbuf, s)

        pltpu.sync_copy(obuf, out_hbm.at[wid])

    fn = pl.kernel(
        body,
        out_type=jax.ShapeDtypeStruct((_NW, opw, _HD), jnp.float32),
        mesh=mesh,
        scratch_types=(
            [pltpu.VMEM((epw,), jnp.int32)] * 4
            + [pltpu.VMEM((epw,), jnp.float32)] * 4
            + [
                pltpu.VMEM((nbuf, 4 * opc * 16, _HD), jnp.bfloat16),
                pltpu.VMEM((opw, _HD), jnp.float32),
            ]
            + [pltpu.SemaphoreType.DMA] * nbuf
        ),
        compiler_params=cp,
    )
    return fn(table, *idxs, *wtss)


def kernel(query, reference_points, input_flatten, W_so, b_so, W_aw, b_aw,
           W_vp, b_vp, W_op, b_op):
    b, lq, d = query.shape
    nq = b * lq

    val = _project(input_flatten.reshape(b * _NT, d), W_vp, b_vp, 1280,
                   out_dtype=jnp.bfloat16)
    table = val.reshape(b * _NT * _NH, _HD)

    i00, i01, i10, i11, w00, w01, w10, w11 = _compute_taps(
        query, reference_points, W_so, b_so, W_aw, b_aw)
    no = nq * _NH
    opw = no // _NW

    sv = _sc_gather(table,
                    [t.reshape(no * 16) for t in (i00, i01, i10, i11)],
                    [t.reshape(no * 16) for t in (w00, w01, w10, w11)],
                    opw)

    out = _project(sv.reshape(nq, d), W_op[_UNPACK_PERM, :], b_op, nq)
    return out.reshape(b, lq, d)


# speedup vs baseline: 1.0060x; 1.0060x over previous
import dataclasses
import functools

import jax
import jax.numpy as jnp
from jax import lax
from jax.experimental import pallas as pl
from jax.experimental.pallas import tpu as pltpu
from jax.experimental.pallas import tpu_sc as plsc

import numpy as np

_NH, _NL, _NP, _HD = 8, 4, 4, 32
_UNPACK_PERM = np.array(
    [h * 32 + (2 * m if m < 16 else 2 * (m - 16) + 1)
     for h in range(8) for m in range(32)], dtype=np.int32)
_SHAPES = ((64, 64), (32, 32), (16, 16), (8, 8))
_NT = sum(h * w for h, w in _SHAPES)
_BASES = (0, 4096, 5120, 5376)
_NW = 32
_TAPS = _NL * _NP * 4
_HP = lax.Precision.HIGHEST


def _matmul_bias_kernel(x_ref, w_ref, b_ref, o_ref):
    o_ref[...] = (
        jnp.dot(x_ref[...], w_ref[...], preferred_element_type=jnp.float32,
                precision=_HP)
        + b_ref[...]
    ).astype(o_ref.dtype)


def _project(x, w, b, block_rows, out_dtype=jnp.float32):
    n, d = x.shape
    dout = w.shape[1]
    return pl.pallas_call(
        _matmul_bias_kernel,
        grid=(n // block_rows,),
        in_specs=[
            pl.BlockSpec((block_rows, d), lambda i: (i, 0)),
            pl.BlockSpec((d, dout), lambda i: (0, 0)),
            pl.BlockSpec((1, dout), lambda i: (0, 0)),
        ],
        out_specs=pl.BlockSpec((block_rows, dout), lambda i: (i, 0)),
        out_shape=jax.ShapeDtypeStruct((n, dout), out_dtype),
    )(x, w, b.reshape(1, dout))


def _sampling_math(q, rx, ry, wso, bso, waw, baw, lq):
    f32 = jnp.float32
    nq = q.shape[0]
    so = jnp.dot(q, wso, preferred_element_type=f32, precision=_HP) + bso
    rr = lax.broadcasted_iota(jnp.int32, (256, 128), 0)
    cc = lax.broadcasted_iota(jnp.int32, (256, 128), 1)
    selx = (rr == 2 * cc).astype(f32)
    sely = (rr == 2 * cc + 1).astype(f32)
    sox = jnp.dot(so, selx, preferred_element_type=f32, precision=_HP)
    soy = jnp.dot(so, sely, preferred_element_type=f32, precision=_HP)
    logits = jnp.dot(q, waw, preferred_element_type=f32, precision=_HP) + baw
    z = jnp.exp(logits)
    r128 = lax.broadcasted_iota(jnp.int32, (128, 128), 0)
    c128 = lax.broadcasted_iota(jnp.int32, (128, 128), 1)
    bd = (r128 // (_NL * _NP) == c128 // (_NL * _NP)).astype(f32)
    aw = z / jnp.dot(z, bd, preferred_element_type=f32, precision=_HP)

    shape = (nq, 128)
    lane = lax.broadcasted_iota(jnp.int32, shape, 1)
    row = lax.broadcasted_iota(jnp.int32, shape, 0)
    lvl = (lane // _NP) % _NL
    h_i = lane // (_NL * _NP)
    b_i = row // lq
    wf = jnp.where(lvl == 0, 64.0, jnp.where(lvl == 1, 32.0,
                   jnp.where(lvl == 2, 16.0, 8.0)))
    base = jnp.where(lvl == 0, _BASES[0], jnp.where(lvl == 1, _BASES[1],
                     jnp.where(lvl == 2, _BASES[2], _BASES[3])))
    cx = jnp.where(lvl == 0, rx[:, 0:1], jnp.where(lvl == 1, rx[:, 1:2],
                   jnp.where(lvl == 2, rx[:, 2:3], rx[:, 3:4])))
    cy = jnp.where(lvl == 0, ry[:, 0:1], jnp.where(lvl == 1, ry[:, 1:2],
                   jnp.where(lvl == 2, ry[:, 2:3], ry[:, 3:4])))
    locx = cx + sox / wf
    locy = cy + soy / wf
    gx = 2.0 * locx - 1.0
    gy = 2.0 * locy - 1.0
    ix = ((gx + 1.0) * wf - 1.0) * 0.5
    iy = ((gy + 1.0) * wf - 1.0) * 0.5
    ix0f = jnp.floor(ix)
    iy0f = jnp.floor(iy)
    fx = ix - ix0f
    fy = iy - iy0f
    ix0 = ix0f.astype(jnp.int32)
    iy0 = iy0f.astype(jnp.int32)
    ix1 = ix0 + 1
    iy1 = iy0 + 1
    wi = wf.astype(jnp.int32)
    vx0 = (ix0 >= 0) & (ix0 < wi)
    vx1 = (ix1 >= 0) & (ix1 < wi)
    vy0 = (iy0 >= 0) & (iy0 < wi)
    vy1 = (iy1 >= 0) & (iy1 < wi)
    m00 = (vx0 & vy0).astype(f32)
    m01 = (vx1 & vy0).astype(f32)
    m10 = (vx0 & vy1).astype(f32)
    m11 = (vx1 & vy1).astype(f32)
    ix0c = jnp.clip(ix0, 0, wi - 1)
    ix1c = jnp.clip(ix1, 0, wi - 1)
    iy0c = jnp.clip(iy0, 0, wi - 1)
    iy1c = jnp.clip(iy1, 0, wi - 1)
    rowbase = b_i * (_NT * _NH) + base * _NH + h_i
    i00 = rowbase + (iy0c * wi + ix0c) * _NH
    i01 = rowbase + (iy0c * wi + ix1c) * _NH
    i10 = rowbase + (iy1c * wi + ix0c) * _NH
    i11 = rowbase + (iy1c * wi + ix1c) * _NH
    ex = 1.0 - fx
    ey = 1.0 - fy
    w00 = ex * ey * m00 * aw
    w01 = fx * ey * m01 * aw
    w10 = ex * fy * m10 * aw
    w11 = fx * fy * m11 * aw
    return i00, i01, i10, i11, w00, w01, w10, w11


def _sampling_kernel(q_ref, rx_ref, ry_ref, wso_ref, bso_ref, waw_ref,
                     baw_ref, *out_refs, lq):
    outs = _sampling_math(q_ref[...], rx_ref[...], ry_ref[...], wso_ref[...],
                          bso_ref[...], waw_ref[...], baw_ref[...], lq)
    for ref, val in zip(out_refs, outs):
        ref[...] = val


def _compute_taps(query, reference_points, W_so, b_so, W_aw, b_aw):
    b, lq, d = query.shape
    nq = b * lq
    q2 = query.reshape(nq, d)
    rx = reference_points[..., 0].reshape(nq, _NL)
    ry = reference_points[..., 1].reshape(nq, _NL)
    ishape = jax.ShapeDtypeStruct((nq, 128), jnp.int32)
    fshape = jax.ShapeDtypeStruct((nq, 128), jnp.float32)
    return pl.pallas_call(
        functools.partial(_sampling_kernel, lq=lq),
        out_shape=[ishape] * 4 + [fshape] * 4,
    )(q2, rx, ry, W_so, b_so.reshape(1, 256), W_aw, b_aw.reshape(1, 128))


def _sc_gather(table, idxs, wtss, opw):
    mesh = plsc.VectorSubcoreMesh(core_axis_name="c", subcore_axis_name="s")
    cp = pltpu.CompilerParams(use_tc_tiling_on_sc=False)
    if "needs_layout_passes" in pltpu.CompilerParams.__dataclass_fields__:
        cp = dataclasses.replace(cp, needs_layout_passes=False)

    nbuf = 3
    opc = 5
    epw = opw * 16
    nchunk = opw // opc
    nrow = opc * 16
    bcast_dnums = lax.GatherDimensionNumbers(
        offset_dims=(), collapsed_slice_dims=(0,), start_index_map=(0,))

    def body(table_hbm, i0_h, i1_h, i2_h, i3_h, w0_h, w1_h, w2_h, w3_h,
             out_hbm, i0, i1, i2, i3, w0, w1, w2, w3, rbuf, obuf, *sems):
        wid = lax.axis_index("s") * 2 + lax.axis_index("c")
        base = wid * epw
        ibufs = (i0, i1, i2, i3)
        wbufs = (w0, w1, w2, w3)
        for src, dst in zip((i0_h, i1_h, i2_h, i3_h), ibufs):
            pltpu.sync_copy(src.at[wid], dst)
        for src, dst in zip((w0_h, w1_h, w2_h, w3_h), wbufs):
            pltpu.sync_copy(src.at[pl.ds(base, epw)], dst)

        def mkcopy(c, t, slot):
            return pltpu.make_async_copy(
                table_hbm.at[ibufs[t].at[c]],
                rbuf.at[slot, pl.ds(t * nrow, nrow)], sems[slot])

        def start(c, slot):
            for t in range(4):
                mkcopy(c, t, slot).start()

        def wait(c, slot):
            for t in range(4):
                mkcopy(c, t, slot).wait()

        def compute(slot, c):
            for k in range(opc):
                o = c * opc + k
                acc0 = jnp.zeros((16,), jnp.float32)
                acc1 = jnp.zeros((16,), jnp.float32)
                for t in range(4):
                    wv = wbufs[t][pl.ds(o * 16, 16)]
                    for j in range(16):
                        wj = lax.gather(
                            wv, jnp.full((16, 1), j, jnp.int32), bcast_dnums,
                            (1,),
                            mode=lax.GatherScatterMode.PROMISE_IN_BOUNDS)
                        r = t * nrow + k * 16 + j
                        row = rbuf[slot, r, pl.ds(0, 32)]
                        v0, v1 = plsc.unpack(
                            row, format=plsc.PackFormat.INTERLEAVED)
                        acc0 = acc0 + wj * v0
                        acc1 = acc1 + wj * v1
                obuf[o, pl.ds(0, 16)] = acc0
                obuf[o, pl.ds(16, 16)] = acc1

        for s in range(nbuf):
            start(s, s)

        @pl.loop(0, nchunk // nbuf)
        def _(i):
            for s in range(nbuf):
                c = i * nbuf + s
                wait(c, s)
                compute(s, c)

                @pl.when(c + nbuf < nchunk)
                def _():
                    start(c + nbuf, s)

        pltpu.sync_copy(obuf, out_hbm.at[wid])

    fn = pl.kernel(
        body,
        out_type=jax.ShapeDtypeStruct((_NW, opw, _HD), jnp.float32),
        mesh=mesh,
        scratch_types=(
            [pltpu.VMEM((nchunk, nrow), jnp.int32)] * 4
            + [pltpu.VMEM((epw,), jnp.float32)] * 4
            + [
                pltpu.VMEM((nbuf, 4 * opc * 16, _HD), jnp.bfloat16),
                pltpu.VMEM((opw, _HD), jnp.float32),
            ]
            + [pltpu.SemaphoreType.DMA] * nbuf
        ),
        compiler_params=cp,
    )
    return fn(table, *idxs, *wtss)


def kernel(query, reference_points, input_flatten, W_so, b_so, W_aw, b_aw,
           W_vp, b_vp, W_op, b_op):
    b, lq, d = query.shape
    nq = b * lq

    val = _project(input_flatten.reshape(b * _NT, d), W_vp, b_vp, 1280,
                   out_dtype=jnp.bfloat16)
    table = val.reshape(b * _NT * _NH, _HD)

    i00, i01, i10, i11, w00, w01, w10, w11 = _compute_taps(
        query, reference_points, W_so, b_so, W_aw, b_aw)
    no = nq * _NH
    opw = no // _NW

    sv = _sc_gather(table,
                    [t.reshape(_NW, opw // 5, 80) for t in (i00, i01, i10, i11)],
                    [t.reshape(no * 16) for t in (w00, w01, w10, w11)],
                    opw)

    out = _project(sv.reshape(nq, d), W_op[_UNPACK_PERM, :], b_op, nq)
    return out.reshape(b, lq, d)


# speedup vs baseline: 1.1867x; 1.1797x over previous
import dataclasses
import functools

import jax
import jax.numpy as jnp
from jax import lax
from jax.experimental import pallas as pl
from jax.experimental.pallas import tpu as pltpu
from jax.experimental.pallas import tpu_sc as plsc

import numpy as np

_NH, _NL, _NP, _HD = 8, 4, 4, 32
_UNPACK_PERM = np.array(
    [h * 32 + (2 * m if m < 16 else 2 * (m - 16) + 1)
     for h in range(8) for m in range(32)], dtype=np.int32)
_SHAPES = ((64, 64), (32, 32), (16, 16), (8, 8))
_NT = sum(h * w for h, w in _SHAPES)
_BASES = (0, 4096, 5120, 5376)
_NW = 32
_TAPS = _NL * _NP * 4
_HP = lax.Precision.HIGHEST


def _matmul_bias_kernel(x_ref, w_ref, b_ref, o_ref, *, precision):
    o_ref[...] = (
        jnp.dot(x_ref[...], w_ref[...], preferred_element_type=jnp.float32,
                precision=precision)
        + b_ref[...]
    ).astype(o_ref.dtype)


def _project(x, w, b, block_rows, out_dtype=jnp.float32,
             precision=lax.Precision.HIGHEST):
    n, d = x.shape
    dout = w.shape[1]
    return pl.pallas_call(
        functools.partial(_matmul_bias_kernel, precision=precision),
        grid=(n // block_rows,),
        in_specs=[
            pl.BlockSpec((block_rows, d), lambda i: (i, 0)),
            pl.BlockSpec((d, dout), lambda i: (0, 0)),
            pl.BlockSpec((1, dout), lambda i: (0, 0)),
        ],
        out_specs=pl.BlockSpec((block_rows, dout), lambda i: (i, 0)),
        out_shape=jax.ShapeDtypeStruct((n, dout), out_dtype),
    )(x, w, b.reshape(1, dout))


def _sampling_math(q, rx, ry, wso, bso, waw, baw, lq):
    f32 = jnp.float32
    nq = q.shape[0]
    so = jnp.dot(q, wso, preferred_element_type=f32, precision=_HP) + bso
    rr = lax.broadcasted_iota(jnp.int32, (256, 128), 0)
    cc = lax.broadcasted_iota(jnp.int32, (256, 128), 1)
    selx = (rr == 2 * cc).astype(f32)
    sely = (rr == 2 * cc + 1).astype(f32)
    sox = jnp.dot(so, selx, preferred_element_type=f32, precision=_HP)
    soy = jnp.dot(so, sely, preferred_element_type=f32, precision=_HP)
    logits = jnp.dot(q, waw, preferred_element_type=f32, precision=_HP) + baw
    z = jnp.exp(logits)
    r128 = lax.broadcasted_iota(jnp.int32, (128, 128), 0)
    c128 = lax.broadcasted_iota(jnp.int32, (128, 128), 1)
    bd = (r128 // (_NL * _NP) == c128 // (_NL * _NP)).astype(f32)
    aw = z / jnp.dot(z, bd, preferred_element_type=f32, precision=_HP)

    shape = (nq, 128)
    lane = lax.broadcasted_iota(jnp.int32, shape, 1)
    row = lax.broadcasted_iota(jnp.int32, shape, 0)
    lvl = (lane // _NP) % _NL
    h_i = lane // (_NL * _NP)
    b_i = row // lq
    wf = jnp.where(lvl == 0, 64.0, jnp.where(lvl == 1, 32.0,
                   jnp.where(lvl == 2, 16.0, 8.0)))
    base = jnp.where(lvl == 0, _BASES[0], jnp.where(lvl == 1, _BASES[1],
                     jnp.where(lvl == 2, _BASES[2], _BASES[3])))
    cx = jnp.where(lvl == 0, rx[:, 0:1], jnp.where(lvl == 1, rx[:, 1:2],
                   jnp.where(lvl == 2, rx[:, 2:3], rx[:, 3:4])))
    cy = jnp.where(lvl == 0, ry[:, 0:1], jnp.where(lvl == 1, ry[:, 1:2],
                   jnp.where(lvl == 2, ry[:, 2:3], ry[:, 3:4])))
    locx = cx + sox / wf
    locy = cy + soy / wf
    gx = 2.0 * locx - 1.0
    gy = 2.0 * locy - 1.0
    ix = ((gx + 1.0) * wf - 1.0) * 0.5
    iy = ((gy + 1.0) * wf - 1.0) * 0.5
    ix0f = jnp.floor(ix)
    iy0f = jnp.floor(iy)
    fx = ix - ix0f
    fy = iy - iy0f
    ix0 = ix0f.astype(jnp.int32)
    iy0 = iy0f.astype(jnp.int32)
    ix1 = ix0 + 1
    iy1 = iy0 + 1
    wi = wf.astype(jnp.int32)
    vx0 = (ix0 >= 0) & (ix0 < wi)
    vx1 = (ix1 >= 0) & (ix1 < wi)
    vy0 = (iy0 >= 0) & (iy0 < wi)
    vy1 = (iy1 >= 0) & (iy1 < wi)
    m00 = (vx0 & vy0).astype(f32)
    m01 = (vx1 & vy0).astype(f32)
    m10 = (vx0 & vy1).astype(f32)
    m11 = (vx1 & vy1).astype(f32)
    ix0c = jnp.clip(ix0, 0, wi - 1)
    ix1c = jnp.clip(ix1, 0, wi - 1)
    iy0c = jnp.clip(iy0, 0, wi - 1)
    iy1c = jnp.clip(iy1, 0, wi - 1)
    rowbase = (b_i * (_NT * _NH) + base * _NH + h_i).astype(f32)
    i00 = rowbase + ((iy0c * wi + ix0c) * _NH).astype(f32)
    i01 = rowbase + ((iy0c * wi + ix1c) * _NH).astype(f32)
    i10 = rowbase + ((iy1c * wi + ix0c) * _NH).astype(f32)
    i11 = rowbase + ((iy1c * wi + ix1c) * _NH).astype(f32)
    pr = lax.broadcasted_iota(jnp.int32, (128, 512), 0)
    pc = lax.broadcasted_iota(jnp.int32, (128, 512), 1)
    slot = (pr // 16) * 64 + (pr % 16)
    iall = jnp.zeros((nq, 512), f32)
    for t, it in enumerate((i00, i01, i10, i11)):
        pt = (pc == slot + t * 16).astype(f32)
        iall = iall + jnp.dot(it, pt, preferred_element_type=f32,
                              precision=_HP)
    ex = 1.0 - fx
    ey = 1.0 - fy
    w00 = ex * ey * m00 * aw
    w01 = fx * ey * m01 * aw
    w10 = ex * fy * m10 * aw
    w11 = fx * fy * m11 * aw
    return jnp.round(iall).astype(jnp.int32), w00, w01, w10, w11


def _sampling_kernel(q_ref, rx_ref, ry_ref, wso_ref, bso_ref, waw_ref,
                     baw_ref, *out_refs, lq):
    outs = _sampling_math(q_ref[...], rx_ref[...], ry_ref[...], wso_ref[...],
                          bso_ref[...], waw_ref[...], baw_ref[...], lq)
    for ref, val in zip(out_refs, outs):
        ref[...] = val


def _compute_taps(query, reference_points, W_so, b_so, W_aw, b_aw):
    b, lq, d = query.shape
    nq = b * lq
    q2 = query.reshape(nq, d)
    rx = reference_points[..., 0].reshape(nq, _NL)
    ry = reference_points[..., 1].reshape(nq, _NL)
    ishape = jax.ShapeDtypeStruct((nq, 512), jnp.int32)
    fshape = jax.ShapeDtypeStruct((nq, 128), jnp.float32)
    return pl.pallas_call(
        functools.partial(_sampling_kernel, lq=lq),
        out_shape=[ishape] + [fshape] * 4,
    )(q2, rx, ry, W_so, b_so.reshape(1, 256), W_aw, b_aw.reshape(1, 128))


def _sc_gather(table, idxs, wtss, opw):
    mesh = plsc.VectorSubcoreMesh(core_axis_name="c", subcore_axis_name="s")
    cp = pltpu.CompilerParams(use_tc_tiling_on_sc=False)
    if "needs_layout_passes" in pltpu.CompilerParams.__dataclass_fields__:
        cp = dataclasses.replace(cp, needs_layout_passes=False)

    nbuf = 3
    opc = 2
    epw = opw * 16
    nchunk = opw // opc
    bcast_dnums = lax.GatherDimensionNumbers(
        offset_dims=(), collapsed_slice_dims=(0,), start_index_map=(0,))

    def body(table_hbm, idx_hbm, w0_h, w1_h, w2_h, w3_h,
             out_hbm, iall, w0, w1, w2, w3, rbuf, obuf, ssem, *sems):
        wid = lax.axis_index("s") * 2 + lax.axis_index("c")
        base = wid * epw
        wbufs = (w0, w1, w2, w3)
        cps = [pltpu.make_async_copy(idx_hbm.at[wid], iall, ssem)]
        cps += [pltpu.make_async_copy(src.at[pl.ds(base, epw)], dst, ssem)
                for src, dst in zip((w0_h, w1_h, w2_h, w3_h), wbufs)]
        for c in cps:
            c.start()
        for c in cps:
            c.wait()

        def mkcopy(c, slot):
            return pltpu.make_async_copy(
                table_hbm.at[iall.at[c]], rbuf.at[slot], sems[slot])

        def compute(slot, c):
            for k in range(opc):
                o = c * opc + k
                acc0 = jnp.zeros((16,), jnp.float32)
                acc1 = jnp.zeros((16,), jnp.float32)
                for t in range(4):
                    wv = wbufs[t][pl.ds(o * 16, 16)]
                    for j in range(16):
                        wj = lax.gather(
                            wv, jnp.full((16, 1), j, jnp.int32), bcast_dnums,
                            (1,),
                            mode=lax.GatherScatterMode.PROMISE_IN_BOUNDS)
                        r = (k * 4 + t) * 16 + j
                        row = rbuf[slot, r, pl.ds(0, 32)]
                        v0, v1 = plsc.unpack(
                            row, format=plsc.PackFormat.INTERLEAVED)
                        acc0 = acc0 + wj * v0
                        acc1 = acc1 + wj * v1
                obuf[o, pl.ds(0, 16)] = acc0
                obuf[o, pl.ds(16, 16)] = acc1

        for s in range(nbuf):
            mkcopy(s, s).start()

        @pl.loop(0, nchunk // nbuf)
        def _(i):
            for s in range(nbuf):
                c = i * nbuf + s
                mkcopy(c, s).wait()
                compute(s, c)

                @pl.when(c + nbuf < nchunk)
                def _():
                    mkcopy(c + nbuf, s).start()

        pltpu.sync_copy(obuf, out_hbm.at[wid])

    fn = pl.kernel(
        body,
        out_type=jax.ShapeDtypeStruct((_NW, opw, _HD), jnp.float32),
        mesh=mesh,
        scratch_types=(
            [pltpu.VMEM((nchunk, opc * 64), jnp.int32)]
            + [pltpu.VMEM((epw,), jnp.float32)] * 4
            + [
                pltpu.VMEM((nbuf, opc * 64, _HD), jnp.bfloat16),
                pltpu.VMEM((opw, _HD), jnp.float32),
                pltpu.SemaphoreType.DMA,
            ]
            + [pltpu.SemaphoreType.DMA] * nbuf
        ),
        compiler_params=cp,
    )
    return fn(table, idxs, *wtss)


def kernel(query, reference_points, input_flatten, W_so, b_so, W_aw, b_aw,
           W_vp, b_vp, W_op, b_op):
    b, lq, d = query.shape
    nq = b * lq

    val = _project(input_flatten.reshape(b * _NT, d), W_vp, b_vp, 1280,
                   out_dtype=jnp.bfloat16, precision=lax.Precision.DEFAULT)
    table = val.reshape(b * _NT * _NH, _HD)

    iall, w00, w01, w10, w11 = _compute_taps(
        query, reference_points, W_so, b_so, W_aw, b_aw)
    no = nq * _NH
    opw = no // _NW

    sv = _sc_gather(table,
                    iall.reshape(_NW, opw // 2, 128),
                    [t.reshape(no * 16) for t in (w00, w01, w10, w11)],
                    opw)

    out = _project(sv.reshape(nq, d), W_op[_UNPACK_PERM, :], b_op, nq)
    return out.reshape(b, lq, d)


# speedup vs baseline: 1.2257x; 1.0329x over previous
import dataclasses
import functools

import jax
import jax.numpy as jnp
from jax import lax
from jax.experimental import pallas as pl
from jax.experimental.pallas import tpu as pltpu
from jax.experimental.pallas import tpu_sc as plsc

import numpy as np

_NH, _NL, _NP, _HD = 8, 4, 4, 32
_UNPACK_PERM = np.array(
    [h * 32 + (2 * m if m < 16 else 2 * (m - 16) + 1)
     for h in range(8) for m in range(32)], dtype=np.int32)
_SHAPES = ((64, 64), (32, 32), (16, 16), (8, 8))
_NT = sum(h * w for h, w in _SHAPES)
_BASES = (0, 4096, 5120, 5376)
_NW = 32
_TAPS = _NL * _NP * 4
_HP = lax.Precision.HIGHEST


def _matmul_bias_kernel(x_ref, w_ref, b_ref, o_ref, *, precision):
    o_ref[...] = (
        jnp.dot(x_ref[...], w_ref[...], preferred_element_type=jnp.float32,
                precision=precision)
        + b_ref[...]
    ).astype(o_ref.dtype)


def _project(x, w, b, block_rows, out_dtype=jnp.float32,
             precision=lax.Precision.HIGHEST):
    n, d = x.shape
    dout = w.shape[1]
    return pl.pallas_call(
        functools.partial(_matmul_bias_kernel, precision=precision),
        grid=(n // block_rows,),
        in_specs=[
            pl.BlockSpec((block_rows, d), lambda i: (i, 0)),
            pl.BlockSpec((d, dout), lambda i: (0, 0)),
            pl.BlockSpec((1, dout), lambda i: (0, 0)),
        ],
        out_specs=pl.BlockSpec((block_rows, dout), lambda i: (i, 0)),
        out_shape=jax.ShapeDtypeStruct((n, dout), out_dtype),
    )(x, w, b.reshape(1, dout))


def _sampling_math(q, rx, ry, wso, bso, waw, baw, lq):
    f32 = jnp.float32
    nq = q.shape[0]
    so = jnp.dot(q, wso, preferred_element_type=f32, precision=_HP) + bso
    rr = lax.broadcasted_iota(jnp.int32, (256, 128), 0)
    cc = lax.broadcasted_iota(jnp.int32, (256, 128), 1)
    selx = (rr == 2 * cc).astype(f32)
    sely = (rr == 2 * cc + 1).astype(f32)
    sox = jnp.dot(so, selx, preferred_element_type=f32, precision=_HP)
    soy = jnp.dot(so, sely, preferred_element_type=f32, precision=_HP)
    logits = jnp.dot(q, waw, preferred_element_type=f32, precision=_HP) + baw
    z = jnp.exp(logits)
    r128 = lax.broadcasted_iota(jnp.int32, (128, 128), 0)
    c128 = lax.broadcasted_iota(jnp.int32, (128, 128), 1)
    bd = (r128 // (_NL * _NP) == c128 // (_NL * _NP)).astype(f32)
    aw = z / jnp.dot(z, bd, preferred_element_type=f32, precision=_HP)

    shape = (nq, 128)
    lane = lax.broadcasted_iota(jnp.int32, shape, 1)
    row = lax.broadcasted_iota(jnp.int32, shape, 0)
    lvl = (lane // _NP) % _NL
    h_i = lane // (_NL * _NP)
    b_i = row // lq
    wf = jnp.where(lvl == 0, 64.0, jnp.where(lvl == 1, 32.0,
                   jnp.where(lvl == 2, 16.0, 8.0)))
    base = jnp.where(lvl == 0, _BASES[0], jnp.where(lvl == 1, _BASES[1],
                     jnp.where(lvl == 2, _BASES[2], _BASES[3])))
    cx = jnp.where(lvl == 0, rx[:, 0:1], jnp.where(lvl == 1, rx[:, 1:2],
                   jnp.where(lvl == 2, rx[:, 2:3], rx[:, 3:4])))
    cy = jnp.where(lvl == 0, ry[:, 0:1], jnp.where(lvl == 1, ry[:, 1:2],
                   jnp.where(lvl == 2, ry[:, 2:3], ry[:, 3:4])))
    locx = cx + sox / wf
    locy = cy + soy / wf
    gx = 2.0 * locx - 1.0
    gy = 2.0 * locy - 1.0
    ix = ((gx + 1.0) * wf - 1.0) * 0.5
    iy = ((gy + 1.0) * wf - 1.0) * 0.5
    ix0f = jnp.floor(ix)
    iy0f = jnp.floor(iy)
    fx = ix - ix0f
    fy = iy - iy0f
    ix0 = ix0f.astype(jnp.int32)
    iy0 = iy0f.astype(jnp.int32)
    ix1 = ix0 + 1
    iy1 = iy0 + 1
    wi = wf.astype(jnp.int32)
    vx0 = (ix0 >= 0) & (ix0 < wi)
    vx1 = (ix1 >= 0) & (ix1 < wi)
    vy0 = (iy0 >= 0) & (iy0 < wi)
    vy1 = (iy1 >= 0) & (iy1 < wi)
    m00 = (vx0 & vy0).astype(f32)
    m01 = (vx1 & vy0).astype(f32)
    m10 = (vx0 & vy1).astype(f32)
    m11 = (vx1 & vy1).astype(f32)
    ix0c = jnp.clip(ix0, 0, wi - 1)
    ix1c = jnp.clip(ix1, 0, wi - 1)
    iy0c = jnp.clip(iy0, 0, wi - 1)
    iy1c = jnp.clip(iy1, 0, wi - 1)
    i00 = (iy0c * wi + ix0c) * _NH
    i01 = (iy0c * wi + ix1c) * _NH
    i10 = (iy1c * wi + ix0c) * _NH
    i11 = (iy1c * wi + ix1c) * _NH
    pr = lax.broadcasted_iota(jnp.int32, (128, 512), 0)
    pc = lax.broadcasted_iota(jnp.int32, (128, 512), 1)
    slot = (pr // 16) * 64 + (pr % 16)
    iall = jnp.zeros((nq, 512), f32)
    for t, it in enumerate((i00, i01, i10, i11)):
        pt = (pc == slot + t * 16).astype(f32)
        hi = (it // 256).astype(f32)
        lo = (it % 256).astype(f32)
        iall = iall + 256.0 * jnp.dot(hi, pt, preferred_element_type=f32,
                                      precision=lax.Precision.DEFAULT)
        iall = iall + jnp.dot(lo, pt, preferred_element_type=f32,
                              precision=lax.Precision.DEFAULT)
    lane2 = lax.broadcasted_iota(jnp.int32, (nq, 512), 1)
    row2 = lax.broadcasted_iota(jnp.int32, (nq, 512), 0)
    lvl2 = (lane2 % 16) // _NP
    base2 = jnp.where(lvl2 == 0, _BASES[0], jnp.where(lvl2 == 1, _BASES[1],
                      jnp.where(lvl2 == 2, _BASES[2], _BASES[3])))
    rowbase2 = ((row2 // lq) * (_NT * _NH) + base2 * _NH
                + lane2 // 64).astype(f32)
    iall = iall + rowbase2
    ex = 1.0 - fx
    ey = 1.0 - fy
    w00 = ex * ey * m00 * aw
    w01 = fx * ey * m01 * aw
    w10 = ex * fy * m10 * aw
    w11 = fx * fy * m11 * aw
    return jnp.round(iall).astype(jnp.int32), w00, w01, w10, w11


def _sampling_kernel(q_ref, rx_ref, ry_ref, wso_ref, bso_ref, waw_ref,
                     baw_ref, *out_refs, lq):
    outs = _sampling_math(q_ref[...], rx_ref[...], ry_ref[...], wso_ref[...],
                          bso_ref[...], waw_ref[...], baw_ref[...], lq)
    for ref, val in zip(out_refs, outs):
        ref[...] = val


def _compute_taps(query, reference_points, W_so, b_so, W_aw, b_aw):
    b, lq, d = query.shape
    nq = b * lq
    q2 = query.reshape(nq, d)
    rx = reference_points[..., 0].reshape(nq, _NL)
    ry = reference_points[..., 1].reshape(nq, _NL)
    ishape = jax.ShapeDtypeStruct((nq, 512), jnp.int32)
    fshape = jax.ShapeDtypeStruct((nq, 128), jnp.float32)
    return pl.pallas_call(
        functools.partial(_sampling_kernel, lq=lq),
        out_shape=[ishape] + [fshape] * 4,
    )(q2, rx, ry, W_so, b_so.reshape(1, 256), W_aw, b_aw.reshape(1, 128))


def _sc_gather(table, idxs, wtss, opw):
    mesh = plsc.VectorSubcoreMesh(core_axis_name="c", subcore_axis_name="s")
    cp = pltpu.CompilerParams(use_tc_tiling_on_sc=False)
    if "needs_layout_passes" in pltpu.CompilerParams.__dataclass_fields__:
        cp = dataclasses.replace(cp, needs_layout_passes=False)

    nbuf = 3
    opc = 2
    epw = opw * 16
    nchunk = opw // opc
    bcast_dnums = lax.GatherDimensionNumbers(
        offset_dims=(), collapsed_slice_dims=(0,), start_index_map=(0,))

    def body(table_hbm, idx_hbm, w0_h, w1_h, w2_h, w3_h,
             out_hbm, iall, w0, w1, w2, w3, rbuf, obuf, ssem, *sems):
        wid = lax.axis_index("s") * 2 + lax.axis_index("c")
        base = wid * epw
        wbufs = (w0, w1, w2, w3)
        cps = [pltpu.make_async_copy(idx_hbm.at[wid], iall, ssem)]
        cps += [pltpu.make_async_copy(src.at[pl.ds(base, epw)], dst, ssem)
                for src, dst in zip((w0_h, w1_h, w2_h, w3_h), wbufs)]
        for c in cps:
            c.start()
        for c in cps:
            c.wait()

        def mkcopy(c, slot):
            return pltpu.make_async_copy(
                table_hbm.at[iall.at[c]], rbuf.at[slot], sems[slot])

        def compute(slot, c):
            for k in range(opc):
                o = c * opc + k
                acc0 = jnp.zeros((16,), jnp.float32)
                acc1 = jnp.zeros((16,), jnp.float32)
                for t in range(4):
                    wv = wbufs[t][pl.ds(o * 16, 16)]
                    for j in range(16):
                        wj = lax.gather(
                            wv, jnp.full((16, 1), j, jnp.int32), bcast_dnums,
                            (1,),
                            mode=lax.GatherScatterMode.PROMISE_IN_BOUNDS)
                        r = (k * 4 + t) * 16 + j
                        row = rbuf[slot, r, pl.ds(0, 32)]
                        v0, v1 = plsc.unpack(
                            row, format=plsc.PackFormat.INTERLEAVED)
                        acc0 = acc0 + wj * v0
                        acc1 = acc1 + wj * v1
                obuf[o, pl.ds(0, 16)] = acc0
                obuf[o, pl.ds(16, 16)] = acc1

        for s in range(nbuf):
            mkcopy(s, s).start()

        @pl.loop(0, nchunk // nbuf)
        def _(i):
            for s in range(nbuf):
                c = i * nbuf + s
                mkcopy(c, s).wait()
                compute(s, c)

                @pl.when(c + nbuf < nchunk)
                def _():
                    mkcopy(c + nbuf, s).start()

        pltpu.sync_copy(obuf, out_hbm.at[wid])

    fn = pl.kernel(
        body,
        out_type=jax.ShapeDtypeStruct((_NW, opw, _HD), jnp.float32),
        mesh=mesh,
        scratch_types=(
            [pltpu.VMEM((nchunk, opc * 64), jnp.int32)]
            + [pltpu.VMEM((epw,), jnp.float32)] * 4
            + [
                pltpu.VMEM((nbuf, opc * 64, _HD), jnp.bfloat16),
                pltpu.VMEM((opw, _HD), jnp.float32),
                pltpu.SemaphoreType.DMA,
            ]
            + [pltpu.SemaphoreType.DMA] * nbuf
        ),
        compiler_params=cp,
    )
    return fn(table, idxs, *wtss)


def kernel(query, reference_points, input_flatten, W_so, b_so, W_aw, b_aw,
           W_vp, b_vp, W_op, b_op):
    b, lq, d = query.shape
    nq = b * lq

    val = _project(input_flatten.reshape(b * _NT, d), W_vp, b_vp, 1280,
                   out_dtype=jnp.bfloat16, precision=lax.Precision.DEFAULT)
    table = val.reshape(b * _NT * _NH, _HD)

    iall, w00, w01, w10, w11 = _compute_taps(
        query, reference_points, W_so, b_so, W_aw, b_aw)
    no = nq * _NH
    opw = no // _NW

    sv = _sc_gather(table,
                    iall.reshape(_NW, opw // 2, 128),
                    [t.reshape(no * 16) for t in (w00, w01, w10, w11)],
                    opw)

    out = _project(sv.reshape(nq, d), W_op[_UNPACK_PERM, :], b_op, nq)
    return out.reshape(b, lq, d)
